# baseline (speedup 1.0000x reference)
import jax
import jax.numpy as jnp
from jax.experimental import pallas as pl

N = 10000
E = 160000
HID = 128
EDGE_DIM = 5
HEADS = 4
EPS_BN = 1e-5


def _bn(x, gamma, beta):
    return x * gamma / jnp.sqrt(1.0 + EPS_BN) + beta


def _combine_body(x0_ref, x1_ref, x2_ref, o_ref):
    o_ref[...] = (x0_ref[...] + x1_ref[...] + x2_ref[...]) * (1.0 / 3.0)


def kernel(x, edge_index, edge_attr, batch, W_in, b_in, g0, be0, W_ge, b_ge, W1, b1, g_m, be_m, W2, b2, Wl, bl, Wr, br, W_e, att, b_gat, g1, be1, g2, be2, s0, s1, Wn1, bn1, Wn2, bn2):
    src, dst = edge_index[0], edge_index[1]
    h = jax.nn.relu(_bn(jnp.dot(x, W_in) + b_in, g0, be0))
    x0 = h
    e = jnp.dot(edge_attr, W_ge) + b_ge
    msg = jax.nn.relu(h[src] + e)
    agg = jax.ops.segment_sum(msg, dst, num_segments=N)
    out = agg + h
    out = jax.nn.relu(_bn(jnp.dot(out, W1) + b1, g_m, be_m))
    out = jnp.dot(out, W2) + b2
    x_new = jax.nn.relu(out)
    sw0 = jax.nn.sigmoid(s0)
    h = sw0 * h + (1.0 - sw0) * x_new
    h = _bn(h, g1, be1)
    x1 = h
    loop_attr = jnp.broadcast_to(jnp.mean(edge_attr, axis=0, keepdims=True), (N, EDGE_DIM))
    ea = jnp.concatenate([edge_attr, loop_attr], axis=0)
    nodes = jnp.arange(N, dtype=src.dtype)
    src2 = jnp.concatenate([src, nodes])
    dst2 = jnp.concatenate([dst, nodes])
    xl = (jnp.dot(h, Wl) + bl).reshape(N, HEADS, HID)
    xr = (jnp.dot(h, Wr) + br).reshape(N, HEADS, HID)
    eatt = jnp.dot(ea, W_e).reshape(-1, HEADS, HID)
    m = jax.nn.leaky_relu(xl[src2] + xr[dst2] + eatt, negative_slope=0.2)
    alpha = jnp.sum(m * att, axis=-1)
    amax = jax.ops.segment_max(alpha, dst2, num_segments=N)
    ex = jnp.exp(alpha - amax[dst2])
    denom = jax.ops.segment_sum(ex, dst2, num_segments=N)
    alpha = ex / (denom[dst2] + 1e-16)
    out = jax.ops.segment_sum(xl[src2] * alpha[:, :, None], dst2, num_segments=N)
    out = jnp.mean(out, axis=1) + b_gat
    x_new = jax.nn.relu(out)
    sw1 = jax.nn.sigmoid(s1)
    h = sw1 * h + (1.0 - sw1) * x_new
    h = _bn(h, g2, be2)
    x2 = h

    final = pl.pallas_call(
        _combine_body,
        out_shape=jax.ShapeDtypeStruct((N, HID), jnp.float32),
        grid=(10,),
        in_specs=[pl.BlockSpec((N // 10, HID), lambda i: (i, 0))] * 3,
        out_specs=pl.BlockSpec((N // 10, HID), lambda i: (i, 0)),
    )(x0, x1, x2)
    return final


# speedup vs baseline: 1.0201x; 1.0201x over previous
import functools

import jax
import jax.numpy as jnp
from jax import lax
from jax.experimental import pallas as pl
from jax.experimental.pallas import tpu as pltpu
from jax.experimental.pallas import tpu_sc as plsc

N = 10000
E = 160000
HID = 128
EDGE_DIM = 5
HEADS = 4
EPS_BN = 1e-5

NC = 2
NS = 16
NW = NC * NS
EPW = E // NW
B = 200
NB = EPW // B
NP = 10240
RPT = NP // NS

_mesh = plsc.VectorSubcoreMesh(core_axis_name="c", subcore_axis_name="s")


@functools.partial(
    pl.kernel,
    mesh=_mesh,
    out_type=jax.ShapeDtypeStruct((NC, NP, HID), jnp.float32),
    scratch_types=[
        pltpu.VMEM((B,), jnp.int32),
        pltpu.VMEM((B,), jnp.int32),
        pltpu.VMEM((B * EDGE_DIM + 16,), jnp.float32),
        pltpu.VMEM((B, HID), jnp.float32),
        pltpu.VMEM((EDGE_DIM * HID,), jnp.float32),
        pltpu.VMEM((HID,), jnp.float32),
        pltpu.VMEM_SHARED((NP, HID), jnp.float32),
        pltpu.SemaphoreType.DMA,
    ],
)
def _gine_edge(h_hbm, src_hbm, dst_hbm, attr_hbm, wge_hbm, bge_hbm, z_hbm,
               out_hbm, src_v, dst_v, attr_v, rows_v, wge_v, bge_v, agg_sh, sem):
    c = lax.axis_index("c")
    s = lax.axis_index("s")
    w = s * NC + c
    pltpu.sync_copy(wge_hbm, wge_v)
    pltpu.sync_copy(bge_hbm, bge_v)
    pltpu.sync_copy(z_hbm, agg_sh.at[pl.ds(s * RPT, RPT)])
    plsc.subcore_barrier()

    base0 = w * EPW

    def blk(b, carry):
        base = base0 + b * B
        pltpu.sync_copy(src_hbm.at[pl.ds(base, B)], src_v)
        pltpu.sync_copy(dst_hbm.at[pl.ds(base, B)], dst_v)
        pltpu.sync_copy(attr_hbm.at[pl.ds(base * EDGE_DIM, B * EDGE_DIM)],
                        attr_v.at[pl.ds(0, B * EDGE_DIM)])
        pltpu.async_copy(h_hbm.at[src_v], rows_v, sem).wait()

        def edge(i, carry2):
            a = attr_v[pl.ds(i * EDGE_DIM, 16)]
            a0, a1, a2, a3, a4 = a[0], a[1], a[2], a[3], a[4]
            for r in range(HID // 16):
                v = rows_v[i, pl.ds(r * 16, 16)]
                e = bge_v[pl.ds(r * 16, 16)]
                e = e + a0 * wge_v[pl.ds(0 * HID + r * 16, 16)]
                e = e + a1 * wge_v[pl.ds(1 * HID + r * 16, 16)]
                e = e + a2 * wge_v[pl.ds(2 * HID + r * 16, 16)]
                e = e + a3 * wge_v[pl.ds(3 * HID + r * 16, 16)]
                e = e + a4 * wge_v[pl.ds(4 * HID + r * 16, 16)]
                rows_v[i, pl.ds(r * 16, 16)] = jnp.maximum(v + e, 0.0)
            return carry2

        lax.fori_loop(0, B, edge, 0)
        pltpu.sync_copy(rows_v, agg_sh.at[dst_v], add=True)
        return carry

    lax.fori_loop(0, NB, blk, 0)
    plsc.subcore_barrier()
    pltpu.sync_copy(agg_sh.at[pl.ds(s * RPT, RPT)],
                    out_hbm.at[c, pl.ds(s * RPT, RPT)])


def _bn(x, gamma, beta):
    return x * gamma / jnp.sqrt(1.0 + EPS_BN) + beta


def _combine_body(x0_ref, x1_ref, x2_ref, o_ref):
    o_ref[...] = (x0_ref[...] + x1_ref[...] + x2_ref[...]) * (1.0 / 3.0)


def kernel(x, edge_index, edge_attr, batch, W_in, b_in, g0, be0, W_ge, b_ge, W1, b1, g_m, be_m, W2, b2, Wl, bl, Wr, br, W_e, att, b_gat, g1, be1, g2, be2, s0, s1, Wn1, bn1, Wn2, bn2):
    src, dst = edge_index[0], edge_index[1]
    h = jax.nn.relu(_bn(jnp.dot(x, W_in) + b_in, g0, be0))
    x0 = h

    zeros = jnp.zeros((RPT, HID), jnp.float32)
    agg2 = _gine_edge(h, src, dst, edge_attr.reshape(-1), W_ge.reshape(-1),
                      b_ge, zeros)
    agg = agg2[0, :N] + agg2[1, :N]

    out = agg + h
    out = jax.nn.relu(_bn(jnp.dot(out, W1) + b1, g_m, be_m))
    out = jnp.dot(out, W2) + b2
    x_new = jax.nn.relu(out)
    sw0 = jax.nn.sigmoid(s0)
    h = sw0 * h + (1.0 - sw0) * x_new
    h = _bn(h, g1, be1)
    x1 = h
    loop_attr = jnp.broadcast_to(jnp.mean(edge_attr, axis=0, keepdims=True), (N, EDGE_DIM))
    ea = jnp.concatenate([edge_attr, loop_attr], axis=0)
    nodes = jnp.arange(N, dtype=src.dtype)
    src2 = jnp.concatenate([src, nodes])
    dst2 = jnp.concatenate([dst, nodes])
    xl = (jnp.dot(h, Wl) + bl).reshape(N, HEADS, HID)
    xr = (jnp.dot(h, Wr) + br).reshape(N, HEADS, HID)
    eatt = jnp.dot(ea, W_e).reshape(-1, HEADS, HID)
    m = jax.nn.leaky_relu(xl[src2] + xr[dst2] + eatt, negative_slope=0.2)
    alpha = jnp.sum(m * att, axis=-1)
    amax = jax.ops.segment_max(alpha, dst2, num_segments=N)
    ex = jnp.exp(alpha - amax[dst2])
    denom = jax.ops.segment_sum(ex, dst2, num_segments=N)
    alpha = ex / (denom[dst2] + 1e-16)
    out = jax.ops.segment_sum(xl[src2] * alpha[:, :, None], dst2, num_segments=N)
    out = jnp.mean(out, axis=1) + b_gat
    x_new = jax.nn.relu(out)
    sw1 = jax.nn.sigmoid(s1)
    h = sw1 * h + (1.0 - sw1) * x_new
    h = _bn(h, g2, be2)
    x2 = h

    final = pl.pallas_call(
        _combine_body,
        out_shape=jax.ShapeDtypeStruct((N, HID), jnp.float32),
        grid=(10,),
        in_specs=[pl.BlockSpec((N // 10, HID), lambda i: (i, 0))] * 3,
        out_specs=pl.BlockSpec((N // 10, HID), lambda i: (i, 0)),
    )(x0, x1, x2)
    return final


# speedup vs baseline: 6.4985x; 6.3706x over previous
import functools
import math

import jax
import jax.numpy as jnp
from jax import lax
from jax.experimental import pallas as pl
from jax.experimental.pallas import tpu as pltpu
from jax.experimental.pallas import tpu_sc as plsc

N = 10000
E = 160000
HID = 128
EDGE_DIM = 5
HEADS = 4
RS = 1.0 / math.sqrt(1.0 + 1e-5)

NC = 2
NS = 16
NW = NC * NS
EPW = E // NW
B = 200
NB = EPW // B
NP = 10240
RPT = NP // NS
BP = 208
NEG = -1e30

_mesh = plsc.VectorSubcoreMesh(core_axis_name="c", subcore_axis_name="s")


def _xlane(v, buf, op):
    for k in (8, 4, 2, 1):
        buf[pl.ds(0, 16)] = v
        buf[pl.ds(16, 16)] = v
        v = op(v, buf[pl.ds(k, 16)])
    return v


def _attr_sum_body(a_ref, o_ref):
    @pl.when(pl.program_id(0) == 0)
    def _():
        o_ref[...] = jnp.zeros_like(o_ref)
    o_ref[...] += jnp.sum(a_ref[...], axis=0, keepdims=True)


def _inproj_body(x_ref, w_ref, b_ref, g_ref, be_ref, o_ref):
    t = jnp.dot(x_ref[...], w_ref[...], preferred_element_type=jnp.float32)
    t = (t + b_ref[...]) * (g_ref[...] * RS) + be_ref[...]
    o_ref[...] = jnp.maximum(t, 0.0)


def _gine_mlp_body(h0_ref, a0_ref, a1_ref, W1_ref, b1_ref, gm_ref, bem_ref,
                   W2_ref, b2_ref, s0_ref, g1_ref, be1_ref, Wl_ref, bl_ref,
                   Wr_ref, br_ref, asum_ref, We_ref, att_ref,
                   h1_ref, xl_ref, xr_ref, as_ref, smax_ref):
    h0 = h0_ref[...]
    outv = a0_ref[...] + a1_ref[...] + h0
    t = jnp.dot(outv, W1_ref[...], preferred_element_type=jnp.float32) + b1_ref[...]
    t = jnp.maximum(t * (gm_ref[...] * RS) + bem_ref[...], 0.0)
    o2 = jnp.dot(t, W2_ref[...], preferred_element_type=jnp.float32) + b2_ref[...]
    xn = jnp.maximum(o2, 0.0)
    sw0 = jax.nn.sigmoid(s0_ref[0, 0])
    h1 = sw0 * h0 + (1.0 - sw0) * xn
    h1 = h1 * (g1_ref[...] * RS) + be1_ref[...]
    h1_ref[...] = h1
    xl = jnp.dot(h1, Wl_ref[...], preferred_element_type=jnp.float32) + bl_ref[...]
    xr = jnp.dot(h1, Wr_ref[...], preferred_element_type=jnp.float32) + br_ref[...]
    xl_ref[...] = xl
    xr_ref[...] = xr
    vloop = jnp.dot(asum_ref[...] * (1.0 / E), We_ref[...],
                    preferred_element_type=jnp.float32)
    m = xl + xr + vloop
    m = jnp.where(m > 0, m, 0.2 * m)
    prod = m * att_ref[...]
    als = jnp.concatenate(
        [jnp.sum(prod[:, h * HID:(h + 1) * HID], axis=1, keepdims=True)
         for h in range(HEADS)], axis=1)
    as_ref[...] = als
    @pl.when(pl.program_id(0) == 0)
    def _():
        smax_ref[...] = jnp.full_like(smax_ref, NEG)
    bm = jnp.max(als, axis=0, keepdims=True)
    smax_ref[...] = jnp.maximum(
        smax_ref[...],
        jnp.concatenate([bm, jnp.full((1, 4), NEG, jnp.float32)], axis=1))


def _c2_body(d0_ref, d1_ref, as_ref, u_ref, invd_ref, ws_ref):
    u = u_ref[0, 0]
    exs = jnp.exp(as_ref[...] - u)
    den = d0_ref[...][:, :HEADS] + d1_ref[...][:, :HEADS] + exs + 1e-16
    inv = 1.0 / den
    invd_ref[...] = inv
    ws_ref[...] = exs * inv


def _final_body(p0_ref, p1_ref, ws_ref, xl_ref, bg_ref, h1_ref, h0_ref,
                s1_ref, g2_ref, be2_ref, o_ref):
    acc = p0_ref[...] + p1_ref[...]
    ws = ws_ref[...]
    xl = xl_ref[...]
    for hh in range(HEADS):
        acc = acc + ws[:, hh:hh + 1] * xl[:, hh * HID:(hh + 1) * HID]
    outg = acc * 0.25 + bg_ref[...]
    xn = jnp.maximum(outg, 0.0)
    sw1 = jax.nn.sigmoid(s1_ref[0, 0])
    h1 = h1_ref[...]
    h2 = sw1 * h1 + (1.0 - sw1) * xn
    h2 = h2 * (g2_ref[...] * RS) + be2_ref[...]
    o_ref[...] = (h0_ref[...] + h1 + h2) * (1.0 / 3.0)


@functools.partial(
    pl.kernel,
    mesh=_mesh,
    compiler_params=pltpu.CompilerParams(needs_layout_passes=False),
    out_type=jax.ShapeDtypeStruct((NC, NP, HID), jnp.float32),
    scratch_types=[
        pltpu.VMEM((B,), jnp.int32),
        pltpu.VMEM((B,), jnp.int32),
        pltpu.VMEM((B * EDGE_DIM + 16,), jnp.float32),
        pltpu.VMEM((B, HID), jnp.float32),
        pltpu.VMEM((EDGE_DIM * HID,), jnp.float32),
        pltpu.VMEM((HID,), jnp.float32),
        pltpu.VMEM_SHARED((NP, HID), jnp.float32),
        pltpu.SemaphoreType.DMA,
    ],
)
def _gine_edge(h_hbm, src_hbm, dst_hbm, attr_hbm, wge_hbm, bge_hbm, z_hbm,
               out_hbm, src_v, dst_v, attr_v, rows_v, wge_v, bge_v, agg_sh, sem):
    c = lax.axis_index("c")
    s = lax.axis_index("s")
    w = s * NC + c
    pltpu.sync_copy(wge_hbm, wge_v)
    pltpu.sync_copy(bge_hbm, bge_v)
    pltpu.sync_copy(z_hbm, agg_sh.at[pl.ds(s * RPT, RPT)])
    plsc.subcore_barrier()

    base0 = w * EPW

    def blk(b, carry):
        base = base0 + b * B
        pltpu.sync_copy(src_hbm.at[pl.ds(base, B)], src_v)
        pltpu.sync_copy(dst_hbm.at[pl.ds(base, B)], dst_v)
        pltpu.sync_copy(attr_hbm.at[pl.ds(base * EDGE_DIM, B * EDGE_DIM)],
                        attr_v.at[pl.ds(0, B * EDGE_DIM)])
        pltpu.async_copy(h_hbm.at[src_v], rows_v, sem).wait()

        def edge(i, carry2):
            a = attr_v[pl.ds(i * EDGE_DIM, 16)]
            a0, a1, a2, a3, a4 = a[0], a[1], a[2], a[3], a[4]
            for r in range(HID // 16):
                v = rows_v[i, pl.ds(r * 16, 16)]
                e = bge_v[pl.ds(r * 16, 16)]
                e = e + a0 * wge_v[pl.ds(0 * HID + r * 16, 16)]
                e = e + a1 * wge_v[pl.ds(1 * HID + r * 16, 16)]
                e = e + a2 * wge_v[pl.ds(2 * HID + r * 16, 16)]
                e = e + a3 * wge_v[pl.ds(3 * HID + r * 16, 16)]
                e = e + a4 * wge_v[pl.ds(4 * HID + r * 16, 16)]
                rows_v[i, pl.ds(r * 16, 16)] = jnp.maximum(v + e, 0.0)
            return carry2

        lax.fori_loop(0, B, edge, 0)
        pltpu.sync_copy(rows_v, agg_sh.at[dst_v], add=True)
        return carry

    lax.fori_loop(0, NB, blk, 0)
    plsc.subcore_barrier()
    pltpu.sync_copy(agg_sh.at[pl.ds(s * RPT, RPT)],
                    out_hbm.at[c, pl.ds(s * RPT, RPT)])


@functools.partial(
    pl.kernel,
    mesh=_mesh,
    compiler_params=pltpu.CompilerParams(needs_layout_passes=False),
    out_type=(jax.ShapeDtypeStruct((HEADS * E,), jnp.float32),
              jax.ShapeDtypeStruct((NW * 64,), jnp.float32)),
    scratch_types=[
        pltpu.VMEM((BP,), jnp.int32),
        pltpu.VMEM((BP,), jnp.int32),
        pltpu.VMEM((BP,), jnp.int32),
        pltpu.VMEM((B * EDGE_DIM + 16,), jnp.float32),
        pltpu.VMEM((BP, HID), jnp.float32),
        pltpu.VMEM((BP, HID), jnp.float32),
        pltpu.VMEM((HEADS * BP,), jnp.float32),
        pltpu.VMEM((EDGE_DIM * HEADS * HID,), jnp.float32),
        pltpu.VMEM((HEADS * HID,), jnp.float32),
        pltpu.VMEM((64,), jnp.float32),
        pltpu.VMEM((32,), jnp.float32),
        pltpu.SemaphoreType.DMA,
    ],
)
def _gat_logits(xl_hbm, xr_hbm, src_hbm, dst_hbm, attr_hbm, we_hbm, att_hbm,
                al_out, mx_out, src_v, dst_v, idx_v, attr_v, xlr_v, xrr_v,
                albuf, we_v, att_v, rmax_v, xbuf, sem):
    c = lax.axis_index("c")
    s = lax.axis_index("s")
    w = s * NC + c
    base0 = w * EPW
    pltpu.sync_copy(we_hbm, we_v)
    pltpu.sync_copy(att_hbm, att_v)
    zi = jnp.zeros((16,), jnp.int32)
    src_v[pl.ds(192, 16)] = zi
    dst_v[pl.ds(192, 16)] = zi
    for h in range(HEADS):
        rmax_v[pl.ds(h * 16, 16)] = jnp.full((16,), NEG, jnp.float32)
        albuf[pl.ds(h * BP + 192, 16)] = jnp.where(
            lax.iota(jnp.int32, 16) < 8, 0.0, NEG).astype(jnp.float32)

    def blk(b, carry):
        base = base0 + b * B
        pltpu.sync_copy(src_hbm.at[pl.ds(base, B)], src_v.at[pl.ds(0, B)])
        pltpu.sync_copy(dst_hbm.at[pl.ds(base, B)], dst_v.at[pl.ds(0, B)])
        pltpu.sync_copy(attr_hbm.at[pl.ds(base * EDGE_DIM, B * EDGE_DIM)],
                        attr_v.at[pl.ds(0, B * EDGE_DIM)])
        for h in range(HEADS):
            def mkidx(j, carry2):
                idx_v[pl.ds(j * 16, 16)] = src_v[pl.ds(j * 16, 16)] * HEADS + h
                return carry2
            lax.fori_loop(0, BP // 16, mkidx, 0)
            pltpu.async_copy(xl_hbm.at[idx_v], xlr_v, sem).wait()

            def mkidx2(j, carry2):
                idx_v[pl.ds(j * 16, 16)] = dst_v[pl.ds(j * 16, 16)] * HEADS + h
                return carry2
            lax.fori_loop(0, BP // 16, mkidx2, 0)
            pltpu.async_copy(xr_hbm.at[idx_v], xrr_v, sem).wait()

            def edge(i, carry2):
                a = attr_v[pl.ds(i * EDGE_DIM, 16)]
                a0, a1, a2, a3, a4 = a[0], a[1], a[2], a[3], a[4]
                acc = jnp.zeros((16,), jnp.float32)
                for r in range(HID // 16):
                    fo = h * HID + r * 16
                    z = xlr_v[i, pl.ds(r * 16, 16)] + xrr_v[i, pl.ds(r * 16, 16)]
                    z = z + a0 * we_v[pl.ds(0 * HEADS * HID + fo, 16)]
                    z = z + a1 * we_v[pl.ds(1 * HEADS * HID + fo, 16)]
                    z = z + a2 * we_v[pl.ds(2 * HEADS * HID + fo, 16)]
                    z = z + a3 * we_v[pl.ds(3 * HEADS * HID + fo, 16)]
                    z = z + a4 * we_v[pl.ds(4 * HEADS * HID + fo, 16)]
                    z = jnp.maximum(z, 0.2 * z)
                    acc = acc + z * att_v[pl.ds(fo, 16)]
                al = _xlane(acc, xbuf, jnp.add)
                plsc.store_scatter(
                    albuf, [jnp.full((16,), h * BP + i, jnp.int32)], al,
                    mask=lax.iota(jnp.int32, 16) == 0)
                return carry2

            lax.fori_loop(0, B, edge, 0)

            def mx(j, carry2):
                rmax_v[pl.ds(h * 16, 16)] = jnp.maximum(
                    rmax_v[pl.ds(h * 16, 16)], albuf[pl.ds(h * BP + j * 16, 16)])
                return carry2
            lax.fori_loop(0, BP // 16, mx, 0)
            pltpu.sync_copy(albuf.at[pl.ds(h * BP, B)],
                            al_out.at[pl.ds(h * E + base, B)])
        return carry

    lax.fori_loop(0, NB, blk, 0)
    pltpu.sync_copy(rmax_v, mx_out.at[pl.ds(w * 64, 64)])


@functools.partial(
    pl.kernel,
    mesh=_mesh,
    compiler_params=pltpu.CompilerParams(needs_layout_passes=False),
    out_type=(jax.ShapeDtypeStruct((HEADS * E,), jnp.float32),
              jax.ShapeDtypeStruct((NC, NP, HID), jnp.float32),
              jax.ShapeDtypeStruct((8,), jnp.float32)),
    scratch_types=[
        pltpu.VMEM((B,), jnp.int32),
        pltpu.VMEM((HEADS * BP,), jnp.float32),
        pltpu.VMEM((HEADS * BP,), jnp.float32),
        pltpu.VMEM((B, HID), jnp.float32),
        pltpu.VMEM((NW * 64,), jnp.float32),
        pltpu.VMEM((16,), jnp.float32),
        pltpu.VMEM((32,), jnp.float32),
        pltpu.VMEM((16,), jnp.float32),
        pltpu.VMEM_SHARED((NP, HID), jnp.float32),
        pltpu.SemaphoreType.DMA,
    ],
)
def _gat_denom(al_hbm, dst_hbm, wmax_hbm, smax_hbm, z_hbm,
               ex_out, dn_out, u_out, dst_v, albuf, exbuf, rowbuf,
               wmax_v, smax_v, xbuf, ubuf, dn_sh, sem):
    c = lax.axis_index("c")
    s = lax.axis_index("s")
    w = s * NC + c
    base0 = w * EPW
    pltpu.sync_copy(z_hbm, dn_sh.at[pl.ds(s * RPT, RPT)])
    pltpu.sync_copy(wmax_hbm, wmax_v)
    smax_v[...] = jnp.full((16,), NEG, jnp.float32)
    pltpu.sync_copy(smax_hbm, smax_v.at[pl.ds(0, 8)])
    mv = smax_v[...]

    def umx(j, m):
        return jnp.maximum(m, wmax_v[pl.ds(j * 16, 16)])
    mv = lax.fori_loop(0, NW * 4, umx, mv)
    uvec = _xlane(mv, xbuf, jnp.maximum)
    u = uvec[0]
    ubuf[...] = jnp.full((16,), 1.0, jnp.float32) * u

    @pl.when(w == 0)
    def _():
        pltpu.sync_copy(ubuf.at[pl.ds(0, 8)], u_out)

    def zr(j, carry):
        for r in range(HID // 16):
            rowbuf[j, pl.ds(r * 16, 16)] = jnp.zeros((16,), jnp.float32)
        return carry
    lax.fori_loop(0, B, zr, 0)
    plsc.subcore_barrier()

    lanes = lax.iota(jnp.int32, 16)

    def blk(b, carry):
        base = base0 + b * B
        pltpu.sync_copy(dst_hbm.at[pl.ds(base, B)], dst_v)
        for h in range(HEADS):
            pltpu.sync_copy(al_hbm.at[pl.ds(h * E + base, B)],
                            albuf.at[pl.ds(h * BP, B)])
        for h in range(HEADS):
            def grp(j, carry2):
                v = jnp.exp(albuf[pl.ds(h * BP + j * 16, 16)] - u)
                exbuf[pl.ds(h * BP + j * 16, 16)] = v
                eidx = lanes + j * 16
                plsc.store_scatter(rowbuf, [eidx, jnp.full((16,), h, jnp.int32)],
                                   v, mask=eidx < B)
                return carry2
            lax.fori_loop(0, BP // 16, grp, 0)
            pltpu.sync_copy(exbuf.at[pl.ds(h * BP, B)],
                            ex_out.at[pl.ds(h * E + base, B)])
        pltpu.sync_copy(rowbuf, dn_sh.at[dst_v], add=True)
        return carry

    lax.fori_loop(0, NB, blk, 0)
    plsc.subcore_barrier()
    pltpu.sync_copy(dn_sh.at[pl.ds(s * RPT, RPT)],
                    dn_out.at[c, pl.ds(s * RPT, RPT)])


@functools.partial(
    pl.kernel,
    mesh=_mesh,
    compiler_params=pltpu.CompilerParams(needs_layout_passes=False),
    out_type=jax.ShapeDtypeStruct((HEADS * E,), jnp.float32),
    scratch_types=[
        pltpu.VMEM((BP,), jnp.int32),
        pltpu.VMEM((HEADS * BP,), jnp.float32),
        pltpu.VMEM((HEADS * BP,), jnp.float32),
        pltpu.VMEM((HEADS * N,), jnp.float32),
    ],
)
def _gat_norm(ex_hbm, dst_hbm, invd_hbm, w_out, dst_v, exb, wb, invd_v):
    c = lax.axis_index("c")
    s = lax.axis_index("s")
    w = s * NC + c
    base0 = w * EPW
    pltpu.sync_copy(invd_hbm, invd_v)
    dst_v[pl.ds(192, 16)] = jnp.zeros((16,), jnp.int32)

    def blk(b, carry):
        base = base0 + b * B
        pltpu.sync_copy(dst_hbm.at[pl.ds(base, B)], dst_v.at[pl.ds(0, B)])
        for h in range(HEADS):
            pltpu.sync_copy(ex_hbm.at[pl.ds(h * E + base, B)],
                            exb.at[pl.ds(h * BP, B)])
            def grp(j, carry2):
                didx = dst_v[pl.ds(j * 16, 16)] * HEADS + h
                iv = plsc.load_gather(invd_v, [didx])
                wb[pl.ds(h * BP + j * 16, 16)] = exb[pl.ds(h * BP + j * 16, 16)] * iv
                return carry2
            lax.fori_loop(0, BP // 16, grp, 0)
            pltpu.sync_copy(wb.at[pl.ds(h * BP, B)],
                            w_out.at[pl.ds(h * E + base, B)])
        return carry

    lax.fori_loop(0, NB, blk, 0)


@functools.partial(
    pl.kernel,
    mesh=_mesh,
    compiler_params=pltpu.CompilerParams(needs_layout_passes=False),
    out_type=jax.ShapeDtypeStruct((NC, NP, HID), jnp.float32),
    scratch_types=[
        pltpu.VMEM((BP,), jnp.int32),
        pltpu.VMEM((B,), jnp.int32),
        pltpu.VMEM((BP,), jnp.int32),
        pltpu.VMEM((BP, HID), jnp.float32),
        pltpu.VMEM((HEADS * BP + 16,), jnp.float32),
        pltpu.VMEM_SHARED((NP, HID), jnp.float32),
        pltpu.SemaphoreType.DMA,
    ],
)
def _gat_agg(xl_hbm, src_hbm, dst_hbm, w_hbm, z_hbm, out_hbm,
             src_v, dst_e, idx_v, xlr_v, wbuf, out_sh, sem):
    c = lax.axis_index("c")
    s = lax.axis_index("s")
    w = s * NC + c
    base0 = w * EPW
    pltpu.sync_copy(z_hbm, out_sh.at[pl.ds(s * RPT, RPT)])
    src_v[pl.ds(192, 16)] = jnp.zeros((16,), jnp.int32)
    plsc.subcore_barrier()

    def blk(b, carry):
        base = base0 + b * B
        pltpu.sync_copy(src_hbm.at[pl.ds(base, B)], src_v.at[pl.ds(0, B)])
        pltpu.sync_copy(dst_hbm.at[pl.ds(base, B)], dst_e)
        for h in range(HEADS):
            pltpu.sync_copy(w_hbm.at[pl.ds(h * E + base, B)],
                            wbuf.at[pl.ds(h * BP, B)])
        for h in range(HEADS):
            def mkidx(j, carry2):
                idx_v[pl.ds(j * 16, 16)] = src_v[pl.ds(j * 16, 16)] * HEADS + h
                return carry2
            lax.fori_loop(0, BP // 16, mkidx, 0)
            pltpu.async_copy(xl_hbm.at[idx_v], xlr_v, sem).wait()

            def edge(i, carry2):
                wv = wbuf[pl.ds(h * BP + i, 16)]
                ws = wv[0]
                for r in range(HID // 16):
                    xlr_v[i, pl.ds(r * 16, 16)] = ws * xlr_v[i, pl.ds(r * 16, 16)]
                return carry2
            lax.fori_loop(0, B, edge, 0)
            pltpu.sync_copy(xlr_v.at[pl.ds(0, B)], out_sh.at[dst_e], add=True)
        return carry

    lax.fori_loop(0, NB, blk, 0)
    plsc.subcore_barrier()
    pltpu.sync_copy(out_sh.at[pl.ds(s * RPT, RPT)],
                    out_hbm.at[c, pl.ds(s * RPT, RPT)])


def kernel(x, edge_index, edge_attr, batch, W_in, b_in, g0, be0, W_ge, b_ge, W1, b1, g_m, be_m, W2, b2, Wl, bl, Wr, br, W_e, att, b_gat, g1, be1, g2, be2, s0, s1, Wn1, bn1, Wn2, bn2):
    f32 = jnp.float32
    src, dst = edge_index[0], edge_index[1]
    attr_flat = edge_attr.reshape(-1)
    z128 = jnp.zeros((RPT, HID), f32)
    row1 = lambda v: v.reshape(1, -1)

    asum = pl.pallas_call(
        _attr_sum_body,
        out_shape=jax.ShapeDtypeStruct((1, EDGE_DIM), f32),
        grid=(20,),
        in_specs=[pl.BlockSpec((E // 20, EDGE_DIM), lambda i: (i, 0))],
        out_specs=pl.BlockSpec((1, EDGE_DIM), lambda i: (0, 0)),
    )(edge_attr)

    h0 = pl.pallas_call(
        _inproj_body,
        out_shape=jax.ShapeDtypeStruct((N, HID), f32),
        grid=(5,),
        in_specs=[
            pl.BlockSpec((N // 5, 10), lambda i: (i, 0)),
            pl.BlockSpec((10, HID), lambda i: (0, 0)),
            pl.BlockSpec((1, HID), lambda i: (0, 0)),
            pl.BlockSpec((1, HID), lambda i: (0, 0)),
            pl.BlockSpec((1, HID), lambda i: (0, 0)),
        ],
        out_specs=pl.BlockSpec((N // 5, HID), lambda i: (i, 0)),
    )(x, W_in, row1(b_in), row1(g0), row1(be0))

    agg2 = _gine_edge(h0, src, dst, attr_flat, W_ge.reshape(-1), b_ge, z128)

    NBK = 10
    R = N // NBK
    rspec = lambda width: pl.BlockSpec((R, width), lambda i: (i, 0))
    wspec = lambda r, cdim: pl.BlockSpec((r, cdim), lambda i: (0, 0))
    h1, xl, xr, alpha_self, smax = pl.pallas_call(
        _gine_mlp_body,
        out_shape=(
            jax.ShapeDtypeStruct((N, HID), f32),
            jax.ShapeDtypeStruct((N, HEADS * HID), f32),
            jax.ShapeDtypeStruct((N, HEADS * HID), f32),
            jax.ShapeDtypeStruct((N, HEADS), f32),
            jax.ShapeDtypeStruct((1, 8), f32),
        ),
        grid=(NBK,),
        in_specs=[
            rspec(HID), rspec(HID), rspec(HID),
            wspec(HID, 2 * HID), wspec(1, 2 * HID), wspec(1, 2 * HID), wspec(1, 2 * HID),
            wspec(2 * HID, HID), wspec(1, HID), wspec(1, 1),
            wspec(1, HID), wspec(1, HID),
            wspec(HID, HEADS * HID), wspec(1, HEADS * HID),
            wspec(HID, HEADS * HID), wspec(1, HEADS * HID),
            wspec(1, EDGE_DIM), wspec(EDGE_DIM, HEADS * HID), wspec(1, HEADS * HID),
        ],
        out_specs=(
            rspec(HID), rspec(HEADS * HID), rspec(HEADS * HID), rspec(HEADS),
            pl.BlockSpec((1, 8), lambda i: (0, 0)),
        ),
    )(h0, agg2[0, :N], agg2[1, :N], W1, row1(b1), row1(g_m), row1(be_m),
      W2, row1(b2), s0.reshape(1, 1), row1(g1), row1(be1),
      Wl, row1(bl), Wr, row1(br), asum, W_e.reshape(EDGE_DIM, -1),
      att.reshape(1, -1))

    xl4 = xl.reshape(N * HEADS, HID)
    xr4 = xr.reshape(N * HEADS, HID)

    alpha_e, wmax = _gat_logits(xl4, xr4, src, dst, attr_flat,
                                W_e.reshape(-1), att.reshape(-1))

    ex_e, dn2, u = _gat_denom(alpha_e, dst, wmax, smax.reshape(-1), z128)

    invd, wself = pl.pallas_call(
        _c2_body,
        out_shape=(jax.ShapeDtypeStruct((N, HEADS), f32),
                   jax.ShapeDtypeStruct((N, HEADS), f32)),
        grid=(NBK,),
        in_specs=[rspec(HID), rspec(HID), rspec(HEADS),
                  pl.BlockSpec((1, 8), lambda i: (0, 0))],
        out_specs=(rspec(HEADS), rspec(HEADS)),
    )(dn2[0, :N], dn2[1, :N], alpha_self, u.reshape(1, 8))

    w_e = _gat_norm(ex_e, dst, invd.reshape(-1))

    gat2 = _gat_agg(xl4, src, dst, w_e, z128)

    final = pl.pallas_call(
        _final_body,
        out_shape=jax.ShapeDtypeStruct((N, HID), f32),
        grid=(NBK,),
        in_specs=[rspec(HID), rspec(HID), rspec(HEADS), rspec(HEADS * HID),
                  wspec(1, HID), rspec(HID), rspec(HID), wspec(1, 1),
                  wspec(1, HID), wspec(1, HID)],
        out_specs=rspec(HID),
    )(gat2[0, :N], gat2[1, :N], wself, xl, row1(b_gat), h1, h0,
      s1.reshape(1, 1), row1(g2), row1(be2))
    return final
